# baseline (speedup 1.0000x reference)
import functools

import jax
import jax.numpy as jnp
from jax import lax
from jax.experimental import pallas as pl
from jax.experimental.pallas import tpu as pltpu
from jax.experimental.pallas import tpu_sc as plsc

_R_EQ6 = 0.5 ** 6
_HALF_DEPTH = 0.238 * 0.5
_CUT = 2.0
_CUT_LO = 1.5

_N_EDGES = 6400000
_N_ATOMS = 100000
_N_MOL = 64

_LANES = 128
_ROWS = _N_EDGES // _LANES
_BLK_ROWS = 1000
_GRID = _ROWS // _BLK_ROWS

_NC = 2
_NS = 16
_NW = _NC * _NS
_EDGES_PER_W = _N_EDGES // _NW
_CHUNK = 4000
_NCHUNK = _EDGES_PER_W // _CHUNK
_VPC = _CHUNK // 16


def _lj_body(v_ref, o_ref):
    w = v_ref[...]
    w = w * w
    k_iota = lax.broadcasted_iota(jnp.int32, (3 * _LANES, _LANES), 0)
    j_iota = lax.broadcasted_iota(jnp.int32, (3 * _LANES, _LANES), 1)
    p = (k_iota // 3 == j_iota).astype(jnp.float32)
    r2 = lax.dot_general(w, p, (((1,), (0,)), ((), ())),
                         preferred_element_type=jnp.float32)
    d = jnp.sqrt(r2)
    inv = 1.0 / r2
    p6 = _R_EQ6 * inv * inv * inv
    yij = p6 * p6 - p6
    u = 2.0 * d - 3.0
    r_function = 1.0 + u * u * (2.0 * u - 3.0)
    switch = jnp.where(d > _CUT_LO, r_function, 1.0)
    switch = jnp.where(d > _CUT, 0.0, switch)
    o_ref[...] = yij * switch


def _lj_energies(v):
    return pl.pallas_call(
        _lj_body,
        grid=(_GRID,),
        in_specs=[pl.BlockSpec((_BLK_ROWS, 3 * _LANES), lambda i: (i, 0))],
        out_specs=pl.BlockSpec((_BLK_ROWS, _LANES), lambda i: (i, 0)),
        out_shape=jax.ShapeDtypeStruct((_ROWS, _LANES), jnp.float32),
    )(v)


def _sc_bin_body(yij_hbm, idxi_hbm, idxm_hbm, out_hbm,
                 table, ybuf, ibuf, acc, binv):
    wid = lax.axis_index("s") * _NC + lax.axis_index("c")
    lane = lax.iota(jnp.int32, 16)
    zero16 = jnp.zeros((16,), jnp.float32)

    pltpu.sync_copy(idxm_hbm, table)

    for m in range(_N_MOL):
        acc[m] = zero16

    def edge_group(j, carry):
        off = j * 16
        iv = ibuf[pl.ds(off, 16)]
        yv = ybuf[pl.ds(off, 16)]
        mol = plsc.load_gather(table, [iv])
        plsc.addupdate_scatter(acc, [mol, lane], yv)
        return carry

    def chunk_body(ch, carry):
        base = pl.multiple_of(wid * _EDGES_PER_W + ch * _CHUNK, _CHUNK)
        pltpu.sync_copy(yij_hbm.at[pl.ds(base, _CHUNK)], ybuf)
        pltpu.sync_copy(idxi_hbm.at[pl.ds(base, _CHUNK)], ibuf)
        lax.fori_loop(0, _VPC, edge_group, carry)
        return carry

    lax.fori_loop(0, _NCHUNK, chunk_body, 0)

    for g in range(4):
        r = zero16
        for t in range(16):
            s = jnp.sum(acc[g * 16 + t])
            r = jnp.where(lane == t, s, r)
        binv[pl.ds(g * 16, 16)] = r

    pltpu.sync_copy(binv, out_hbm.at[wid])


def _sc_bin(yij_flat, idx_i, idx_m):
    f = pl.kernel(
        _sc_bin_body,
        out_type=jax.ShapeDtypeStruct((_NW, _N_MOL), jnp.float32),
        mesh=plsc.VectorSubcoreMesh(core_axis_name="c", subcore_axis_name="s"),
        compiler_params=pltpu.CompilerParams(needs_layout_passes=False),
        scratch_types=[
            pltpu.VMEM((_N_ATOMS,), jnp.int32),
            pltpu.VMEM((_CHUNK,), jnp.float32),
            pltpu.VMEM((_CHUNK,), jnp.int32),
            pltpu.VMEM((_N_MOL, 16), jnp.float32),
            pltpu.VMEM((_N_MOL,), jnp.float32),
        ],
    )
    return f(yij_flat, idx_i, idx_m)


def _finish_body(b_ref, o_ref):
    o_ref[...] = jnp.sum(b_ref[...], axis=0, keepdims=True) * _HALF_DEPTH


def _finish(partials):
    return pl.pallas_call(
        _finish_body,
        out_shape=jax.ShapeDtypeStruct((1, _N_MOL), jnp.float32),
    )(partials)


def kernel(Rij, R, idx_i, idx_m):
    v = Rij.reshape(_ROWS, 3 * _LANES)
    yij = _lj_energies(v)
    partials = _sc_bin(yij.reshape(_N_EDGES), idx_i, idx_m)
    y = _finish(partials)
    return y.reshape(_N_MOL)


# speedup vs baseline: 23.4958x; 23.4958x over previous
import functools

import jax
import jax.numpy as jnp
from jax import lax
from jax.experimental import pallas as pl
from jax.experimental.pallas import tpu as pltpu
from jax.experimental.pallas import tpu_sc as plsc

_R_EQ6 = 0.5 ** 6
_HALF_DEPTH = 0.238 * 0.5
_CUT = 2.0
_CUT_LO = 1.5

_N_EDGES = 6400000
_N_ATOMS = 100000
_N_MOL = 64

_LANES = 128
_ROWS = _N_EDGES // _LANES
_BLK_ROWS = 1000
_GRID = _ROWS // _BLK_ROWS

_NC = 2
_NS = 16
_NW = _NC * _NS
_EDGES_PER_W = _N_EDGES // _NW
_CHUNK = 4000
_NCHUNK = _EDGES_PER_W // _CHUNK
_VPC = _CHUNK // 16


def _lj_body(v_ref, o_ref):
    x = v_ref[0, :]
    y = v_ref[1, :]
    z = v_ref[2, :]
    r2 = x * x + y * y + z * z
    d = jnp.sqrt(r2)
    inv = 1.0 / r2
    p6 = _R_EQ6 * inv * inv * inv
    yij = p6 * p6 - p6
    u = 2.0 * d - 3.0
    r_function = 1.0 + u * u * (2.0 * u - 3.0)
    switch = jnp.where(d > _CUT_LO, r_function, 1.0)
    switch = jnp.where(d > _CUT, 0.0, switch)
    o_ref[...] = yij * switch


_BLK_E = 128000
_GRID_E = _N_EDGES // _BLK_E


def _lj_energies(xt):
    return pl.pallas_call(
        _lj_body,
        grid=(_GRID_E,),
        in_specs=[pl.BlockSpec((3, _BLK_E), lambda i: (0, i))],
        out_specs=pl.BlockSpec((_BLK_E,), lambda i: (i,)),
        out_shape=jax.ShapeDtypeStruct((_N_EDGES,), jnp.float32),
        compiler_params=pltpu.CompilerParams(allow_input_fusion=[True]),
    )(xt)


def _sc_bin_body(yij_hbm, idxi_hbm, idxm_hbm, out_hbm,
                 table, ybuf, ibuf, acc, binv):
    wid = lax.axis_index("s") * _NC + lax.axis_index("c")
    lane = lax.iota(jnp.int32, 16)
    zero16 = jnp.zeros((16,), jnp.float32)

    pltpu.sync_copy(idxm_hbm, table)

    for m in range(_N_MOL):
        acc[m] = zero16

    def edge_group(j, carry):
        off = j * 16
        iv = ibuf[pl.ds(off, 16)]
        yv = ybuf[pl.ds(off, 16)]
        mol = plsc.load_gather(table, [iv])
        plsc.addupdate_scatter(acc, [mol, lane], yv)
        return carry

    def chunk_body(ch, carry):
        base = pl.multiple_of(wid * _EDGES_PER_W + ch * _CHUNK, _CHUNK)
        pltpu.sync_copy(yij_hbm.at[pl.ds(base, _CHUNK)], ybuf)
        pltpu.sync_copy(idxi_hbm.at[pl.ds(base, _CHUNK)], ibuf)
        lax.fori_loop(0, _VPC, edge_group, carry)
        return carry

    lax.fori_loop(0, _NCHUNK, chunk_body, 0)

    for g in range(4):
        r = zero16
        for t in range(16):
            s = jnp.sum(acc[g * 16 + t])
            r = jnp.where(lane == t, s, r)
        binv[pl.ds(g * 16, 16)] = r

    pltpu.sync_copy(binv, out_hbm.at[wid])


def _sc_bin(yij_flat, idx_i, idx_m):
    f = pl.kernel(
        _sc_bin_body,
        out_type=jax.ShapeDtypeStruct((_NW, _N_MOL), jnp.float32),
        mesh=plsc.VectorSubcoreMesh(core_axis_name="c", subcore_axis_name="s"),
        compiler_params=pltpu.CompilerParams(needs_layout_passes=False),
        scratch_types=[
            pltpu.VMEM((_N_ATOMS,), jnp.int32),
            pltpu.VMEM((_CHUNK,), jnp.float32),
            pltpu.VMEM((_CHUNK,), jnp.int32),
            pltpu.VMEM((_N_MOL, 16), jnp.float32),
            pltpu.VMEM((_N_MOL,), jnp.float32),
        ],
    )
    return f(yij_flat, idx_i, idx_m)


def _finish_body(b_ref, o_ref):
    o_ref[...] = jnp.sum(b_ref[...], axis=0, keepdims=True) * _HALF_DEPTH


def _finish(partials):
    return pl.pallas_call(
        _finish_body,
        out_shape=jax.ShapeDtypeStruct((1, _N_MOL), jnp.float32),
    )(partials)


def kernel(Rij, R, idx_i, idx_m):
    yij = _lj_energies(Rij.T)
    partials = _sc_bin(yij, idx_i, idx_m)
    y = _finish(partials)
    return y.reshape(_N_MOL)


# speedup vs baseline: 31.5579x; 1.3431x over previous
import functools

import jax
import jax.numpy as jnp
from jax import lax
from jax.experimental import pallas as pl
from jax.experimental.pallas import tpu as pltpu
from jax.experimental.pallas import tpu_sc as plsc

_R_EQ6 = 0.5 ** 6
_HALF_DEPTH = 0.238 * 0.5
_CUT = 2.0
_CUT_LO = 1.5

_N_EDGES = 6400000
_N_ATOMS = 100000
_N_MOL = 64

_LANES = 128
_ROWS = _N_EDGES // _LANES
_BLK_ROWS = 1000
_GRID = _ROWS // _BLK_ROWS

_NC = 2
_NS = 16
_NW = _NC * _NS
_EDGES_PER_W = _N_EDGES // _NW
_CHUNK = 4000
_NCHUNK = _EDGES_PER_W // _CHUNK
_VPC = _CHUNK // 16


def _lj_body(v_ref, o_ref):
    x = v_ref[0, :]
    y = v_ref[1, :]
    z = v_ref[2, :]
    r2 = x * x + y * y + z * z
    d = jnp.sqrt(r2)
    inv = 1.0 / r2
    p6 = _R_EQ6 * inv * inv * inv
    yij = p6 * p6 - p6
    u = 2.0 * d - 3.0
    r_function = 1.0 + u * u * (2.0 * u - 3.0)
    switch = jnp.where(d > _CUT_LO, r_function, 1.0)
    switch = jnp.where(d > _CUT, 0.0, switch)
    o_ref[...] = yij * switch


_BLK_E = 128000
_GRID_E = _N_EDGES // _BLK_E


def _lj_energies(xt):
    return pl.pallas_call(
        _lj_body,
        grid=(_GRID_E,),
        in_specs=[pl.BlockSpec((3, _BLK_E), lambda i: (0, i))],
        out_specs=pl.BlockSpec((_BLK_E,), lambda i: (i,)),
        out_shape=jax.ShapeDtypeStruct((_N_EDGES,), jnp.float32),
        compiler_params=pltpu.CompilerParams(allow_input_fusion=[True]),
    )(xt)


_UNROLL = 4


def _sc_bin_body(yij_hbm, idxi_hbm, idxm_hbm, out_hbm,
                 table, ybuf0, ibuf0, ybuf1, ibuf1, acc, binv,
                 sy0, si0, sy1, si1):
    wid = lax.axis_index("s") * _NC + lax.axis_index("c")
    lane = lax.iota(jnp.int32, 16)
    zero16 = jnp.zeros((16,), jnp.float32)
    base_w = pl.multiple_of(wid * _EDGES_PER_W, _EDGES_PER_W)

    pltpu.sync_copy(idxm_hbm, table)

    for m in range(_N_MOL):
        acc[m] = zero16

    def start(ch, yb, ib, sy, si):
        base = pl.multiple_of(base_w + ch * _CHUNK, _CHUNK)
        pltpu.async_copy(yij_hbm.at[pl.ds(base, _CHUNK)], yb, sy)
        pltpu.async_copy(idxi_hbm.at[pl.ds(base, _CHUNK)], ib, si)

    def wait(ch, yb, ib, sy, si):
        base = pl.multiple_of(base_w + ch * _CHUNK, _CHUNK)
        pltpu.make_async_copy(yij_hbm.at[pl.ds(base, _CHUNK)], yb, sy).wait()
        pltpu.make_async_copy(idxi_hbm.at[pl.ds(base, _CHUNK)], ib, si).wait()

    def process(yb, ib):
        def edge_group(j, carry):
            off = j * (16 * _UNROLL)
            for u in range(_UNROLL):
                iv = ib[pl.ds(off + 16 * u, 16)]
                yv = yb[pl.ds(off + 16 * u, 16)]
                mol = plsc.load_gather(table, [iv])
                plsc.addupdate_scatter(acc, [mol, lane], yv)
            return carry
        lax.fori_loop(0, _VPC // _UNROLL, edge_group, 0)

    start(0, ybuf0, ibuf0, sy0, si0)

    def pair_body(p, carry):
        ch0 = p * 2
        start(ch0 + 1, ybuf1, ibuf1, sy1, si1)
        wait(ch0, ybuf0, ibuf0, sy0, si0)
        process(ybuf0, ibuf0)

        @pl.when(p < _NCHUNK // 2 - 1)
        def _():
            start(ch0 + 2, ybuf0, ibuf0, sy0, si0)

        wait(ch0 + 1, ybuf1, ibuf1, sy1, si1)
        process(ybuf1, ibuf1)
        return carry

    lax.fori_loop(0, _NCHUNK // 2, pair_body, 0)

    for g in range(4):
        r = zero16
        for t in range(16):
            s = jnp.sum(acc[g * 16 + t])
            r = jnp.where(lane == t, s, r)
        binv[pl.ds(g * 16, 16)] = r

    pltpu.sync_copy(binv, out_hbm.at[wid])


def _sc_bin(yij_flat, idx_i, idx_m):
    f = pl.kernel(
        _sc_bin_body,
        out_type=jax.ShapeDtypeStruct((_NW, _N_MOL), jnp.float32),
        mesh=plsc.VectorSubcoreMesh(core_axis_name="c", subcore_axis_name="s"),
        compiler_params=pltpu.CompilerParams(needs_layout_passes=False),
        scratch_types=[
            pltpu.VMEM((_N_ATOMS,), jnp.int32),
            pltpu.VMEM((_CHUNK,), jnp.float32),
            pltpu.VMEM((_CHUNK,), jnp.int32),
            pltpu.VMEM((_CHUNK,), jnp.float32),
            pltpu.VMEM((_CHUNK,), jnp.int32),
            pltpu.VMEM((_N_MOL, 16), jnp.float32),
            pltpu.VMEM((_N_MOL,), jnp.float32),
            pltpu.SemaphoreType.DMA,
            pltpu.SemaphoreType.DMA,
            pltpu.SemaphoreType.DMA,
            pltpu.SemaphoreType.DMA,
        ],
    )
    return f(yij_flat, idx_i, idx_m)


def _finish_body(b_ref, o_ref):
    o_ref[...] = jnp.sum(b_ref[...], axis=0, keepdims=True) * _HALF_DEPTH


def _finish(partials):
    return pl.pallas_call(
        _finish_body,
        out_shape=jax.ShapeDtypeStruct((1, _N_MOL), jnp.float32),
    )(partials)


def kernel(Rij, R, idx_i, idx_m):
    yij = _lj_energies(Rij.T)
    partials = _sc_bin(yij, idx_i, idx_m)
    y = _finish(partials)
    return y.reshape(_N_MOL)


# speedup vs baseline: 56.5736x; 1.7927x over previous
import functools

import jax
import jax.numpy as jnp
from jax import lax
from jax.experimental import pallas as pl
from jax.experimental.pallas import tpu as pltpu
from jax.experimental.pallas import tpu_sc as plsc

_R_EQ6 = 0.5 ** 6
_HALF_DEPTH = 0.238 * 0.5
_CUT = 2.0
_CUT_LO = 1.5

_N_EDGES = 6400000
_N_ATOMS = 100000
_N_MOL = 64

_LANES = 128
_ROWS = _N_EDGES // _LANES
_BLK_ROWS = 1000
_GRID = _ROWS // _BLK_ROWS

_NC = 2
_NS = 16
_NW = _NC * _NS
_EDGES_PER_W = _N_EDGES // _NW
_CHUNK = 4000
_NCHUNK = _EDGES_PER_W // _CHUNK
_VPC = _CHUNK // 16


def _lj_body(v_ref, o_ref):
    x = v_ref[0, :]
    y = v_ref[1, :]
    z = v_ref[2, :]
    r2 = x * x + y * y + z * z
    d = jnp.sqrt(r2)
    inv = 1.0 / r2
    p6 = _R_EQ6 * inv * inv * inv
    yij = p6 * p6 - p6
    u = 2.0 * d - 3.0
    r_function = 1.0 + u * u * (2.0 * u - 3.0)
    switch = jnp.where(d > _CUT_LO, r_function, 1.0)
    switch = jnp.where(d > _CUT, 0.0, switch)
    o_ref[...] = yij * switch


_BLK_E = 128000
_GRID_E = _N_EDGES // _BLK_E


def _lj_energies(xt):
    return pl.pallas_call(
        _lj_body,
        grid=(_GRID_E,),
        in_specs=[pl.BlockSpec((3, _BLK_E), lambda i: (0, i))],
        out_specs=pl.BlockSpec((_BLK_E,), lambda i: (i,)),
        out_shape=jax.ShapeDtypeStruct((_N_EDGES,), jnp.float32),
        compiler_params=pltpu.CompilerParams(allow_input_fusion=[True]),
    )(xt)


_UNROLL = 4


def _sc_bin_body(yij_hbm, idxi_hbm, idxm_hbm, out_hbm,
                 table, ybuf0, ibuf0, ybuf1, ibuf1, acc, binv,
                 sy0, si0, sy1, si1):
    wid = lax.axis_index("s") * _NC + lax.axis_index("c")
    lane = lax.iota(jnp.int32, 16)
    zero16 = jnp.zeros((16,), jnp.float32)
    base_w = pl.multiple_of(wid * _EDGES_PER_W, _EDGES_PER_W)

    pltpu.sync_copy(idxm_hbm, table)

    for m in range(_N_MOL):
        acc[m] = zero16

    def start(ch, yb, ib, sy, si):
        base = pl.multiple_of(base_w + ch * _CHUNK, _CHUNK)
        pltpu.async_copy(yij_hbm.at[pl.ds(base, _CHUNK)], yb, sy)
        pltpu.async_copy(idxi_hbm.at[pl.ds(base, _CHUNK)], ib, si)

    def wait(ch, yb, ib, sy, si):
        base = pl.multiple_of(base_w + ch * _CHUNK, _CHUNK)
        pltpu.make_async_copy(yij_hbm.at[pl.ds(base, _CHUNK)], yb, sy).wait()
        pltpu.make_async_copy(idxi_hbm.at[pl.ds(base, _CHUNK)], ib, si).wait()

    def process(yb, ib):
        @plsc.parallel_loop(0, _CHUNK, step=16, unroll=_UNROLL)
        def _(off):
            iv = ib[pl.ds(off, 16)]
            yv = yb[pl.ds(off, 16)]
            mol = plsc.load_gather(table, [iv])
            plsc.addupdate_scatter(acc, [mol, lane], yv)

    start(0, ybuf0, ibuf0, sy0, si0)

    def pair_body(p, carry):
        ch0 = p * 2
        start(ch0 + 1, ybuf1, ibuf1, sy1, si1)
        wait(ch0, ybuf0, ibuf0, sy0, si0)
        process(ybuf0, ibuf0)

        @pl.when(p < _NCHUNK // 2 - 1)
        def _():
            start(ch0 + 2, ybuf0, ibuf0, sy0, si0)

        wait(ch0 + 1, ybuf1, ibuf1, sy1, si1)
        process(ybuf1, ibuf1)
        return carry

    lax.fori_loop(0, _NCHUNK // 2, pair_body, 0)

    for g in range(4):
        r = zero16
        for t in range(16):
            s = jnp.sum(acc[g * 16 + t])
            r = jnp.where(lane == t, s, r)
        binv[pl.ds(g * 16, 16)] = r

    pltpu.sync_copy(binv, out_hbm.at[wid])


def _sc_bin(yij_flat, idx_i, idx_m):
    f = pl.kernel(
        _sc_bin_body,
        out_type=jax.ShapeDtypeStruct((_NW, _N_MOL), jnp.float32),
        mesh=plsc.VectorSubcoreMesh(core_axis_name="c", subcore_axis_name="s"),
        compiler_params=pltpu.CompilerParams(needs_layout_passes=False),
        scratch_types=[
            pltpu.VMEM((_N_ATOMS,), jnp.int32),
            pltpu.VMEM((_CHUNK,), jnp.float32),
            pltpu.VMEM((_CHUNK,), jnp.int32),
            pltpu.VMEM((_CHUNK,), jnp.float32),
            pltpu.VMEM((_CHUNK,), jnp.int32),
            pltpu.VMEM((_N_MOL, 16), jnp.float32),
            pltpu.VMEM((_N_MOL,), jnp.float32),
            pltpu.SemaphoreType.DMA,
            pltpu.SemaphoreType.DMA,
            pltpu.SemaphoreType.DMA,
            pltpu.SemaphoreType.DMA,
        ],
    )
    return f(yij_flat, idx_i, idx_m)


def _finish_body(b_ref, o_ref):
    o_ref[...] = jnp.sum(b_ref[...], axis=0, keepdims=True) * _HALF_DEPTH


def _finish(partials):
    return pl.pallas_call(
        _finish_body,
        out_shape=jax.ShapeDtypeStruct((1, _N_MOL), jnp.float32),
    )(partials)


def kernel(Rij, R, idx_i, idx_m):
    yij = _lj_energies(Rij.T)
    partials = _sc_bin(yij, idx_i, idx_m)
    y = _finish(partials)
    return y.reshape(_N_MOL)
